# baseline (speedup 1.0000x reference)
import functools

import jax
import jax.numpy as jnp
from jax import lax
from jax.experimental import pallas as pl
from jax.experimental.pallas import tpu as pltpu
from jax.experimental.pallas import tpu_sc as plsc

N_NODES = 10000
D = 128
N_EDGES = 320000
B = 8192
P = 20

NC = 2
NS = 16
NW = NC * NS

EPW = N_EDGES // NW
ECH = 128
NCHP = 80
EPWP = NCHP * ECH
ICH = 16
TRASH = 10016

NPAD = 10240
STRIDE = NPAD // NS

BPW = B // NW
G = 4
NG = BPW // G

_f32 = jnp.float32


def _edge_body(x_hbm, srcL_hbm, dstL_hbm,
               msum_hbm,
               sidx_v, didx_v, rows_a, rows_b,
               msum_sh, sem_a, sem_b):
    c = lax.axis_index("c")
    s = lax.axis_index("s")
    wid = s * NC + c

    zero16 = jnp.zeros((16,), _f32)

    def zrow(i, _):
        r = i // (D // 16)
        col = (i % (D // 16)) * 16
        rows_a[r, pl.ds(col, 16)] = zero16
        return 0

    lax.fori_loop(0, ECH * (D // 16), zrow, 0)

    for k in range(STRIDE // ECH):
        off = s * STRIDE + k * ECH
        pltpu.sync_copy(rows_a, msum_sh.at[pl.ds(off, ECH)])
    plsc.subcore_barrier()

    rows = (rows_a, rows_b)
    sems = (sem_a, sem_b)
    for blk in range(NCHP // ICH):
        pltpu.sync_copy(srcL_hbm.at[wid, pl.ds(blk * ICH, ICH)], sidx_v)
        pltpu.sync_copy(dstL_hbm.at[wid, pl.ds(blk * ICH, ICH)], didx_v)
        pend = pltpu.async_copy(x_hbm.at[sidx_v.at[0]], rows[0], sems[0])
        for j in range(ICH):
            if j + 1 < ICH:
                nxt = pltpu.async_copy(x_hbm.at[sidx_v.at[j + 1]],
                                       rows[(j + 1) % 2], sems[(j + 1) % 2])
            pend.wait()
            pltpu.sync_copy(rows[j % 2], msum_sh.at[didx_v.at[j]], add=True)
            if j + 1 < ICH:
                pend = nxt
    plsc.subcore_barrier()

    for k in range(STRIDE // ECH):
        off = s * STRIDE + k * ECH
        pltpu.sync_copy(msum_sh.at[pl.ds(off, ECH)], rows[k % 2])
        pltpu.sync_copy(rows[k % 2], msum_hbm.at[c, pl.ds(off, ECH)])


_edge_call = functools.partial(
    pl.kernel,
    out_type=jax.ShapeDtypeStruct((NC, NPAD, D), _f32),
    mesh=plsc.VectorSubcoreMesh(core_axis_name="c", subcore_axis_name="s",
                                num_cores=NC, num_subcores=NS),
    compiler_params=pltpu.CompilerParams(needs_layout_passes=False),
    scratch_types=[
        pltpu.VMEM((ICH, ECH), jnp.int32),
        pltpu.VMEM((ICH, ECH), jnp.int32),
        pltpu.VMEM((ECH, D), _f32),
        pltpu.VMEM((ECH, D), _f32),
        pltpu.VMEM_SHARED((NPAD, D), _f32),
        pltpu.SemaphoreType.DMA,
        pltpu.SemaphoreType.DMA,
    ],
)(_edge_body)


def _deg_body(dstF_hbm, dcnt_hbm, didx_v, deg_v):
    c = lax.axis_index("c")
    s = lax.axis_index("s")
    wid = s * NC + c

    pltpu.sync_copy(dstF_hbm.at[wid], didx_v)

    zero16 = jnp.zeros((16,), _f32)
    one16 = jnp.ones((16,), _f32)

    def zr(i, _):
        deg_v[pl.ds(i * 16, 16)] = zero16
        return 0

    lax.fori_loop(0, NPAD // 16, zr, 0)

    def chunk(i, _):
        idx = didx_v[pl.ds(i * 16, 16)]
        plsc.addupdate_scatter(deg_v, [idx], one16)
        return 0

    lax.fori_loop(0, EPW // 16, chunk, 0)

    pltpu.sync_copy(deg_v, dcnt_hbm.at[wid])


_deg_call = functools.partial(
    pl.kernel,
    out_type=jax.ShapeDtypeStruct((NW, NPAD), _f32),
    mesh=plsc.VectorSubcoreMesh(core_axis_name="c", subcore_axis_name="s",
                                num_cores=NC, num_subcores=NS),
    compiler_params=pltpu.CompilerParams(needs_layout_passes=False),
    scratch_types=[
        pltpu.VMEM((EPW,), jnp.int32),
        pltpu.VMEM((NPAD,), _f32),
    ],
)(_deg_body)


RBLK = 1024


def _h_body(x_ref, m0_ref, m1_ref, d_ref, ws_ref, wn_ref, b_ref,
            o_ref):
    i = pl.program_id(0)
    deg = jnp.sum(d_ref[...], axis=0)[:, None]
    agg = (m0_ref[...] + m1_ref[...]) / jnp.maximum(deg, 1.0)
    h = jnp.dot(x_ref[...], ws_ref[...], preferred_element_type=_f32)
    h = h + jnp.dot(agg, wn_ref[...], preferred_element_type=_f32)
    h = jnp.maximum(h + b_ref[...], 0.0)
    rows = i * RBLK + lax.broadcasted_iota(jnp.int32, (RBLK, D), 0)
    o_ref[...] = jnp.where(rows < N_NODES, h, 0.0)


_h_call = pl.pallas_call(
    _h_body,
    grid=(NPAD // RBLK,),
    in_specs=[
        pl.BlockSpec((RBLK, D), lambda i: (i, 0)),
        pl.BlockSpec((RBLK, D), lambda i: (i, 0)),
        pl.BlockSpec((RBLK, D), lambda i: (i, 0)),
        pl.BlockSpec((NW, RBLK), lambda i: (0, i)),
        pl.BlockSpec((D, D), lambda i: (0, 0)),
        pl.BlockSpec((D, D), lambda i: (0, 0)),
        pl.BlockSpec((1, D), lambda i: (0, 0)),
    ],
    out_specs=pl.BlockSpec((RBLK, D), lambda i: (i, 0)),
    out_shape=jax.ShapeDtypeStruct((NPAD, D), _f32),
)


NB_PAD = 10008
GE = 16
HGE = 8
NBLK = BPW // GE
HLEN = 2 * HGE + 2 * HGE * P
QI = HLEN // 3
GIDX_W = NBLK * 2 * HLEN


def _score_body(h_hbm, gidx_hbm, src_hbm, dst_hbm, cs_hbm, cd_hbm,
                nb_hbm, mu_hbm, score_hbm,
                gidx_v, src_v, dst_v, cs_v, cd_v, nb_v, mu_v,
                rows_a, rows_b, out_v, sem_a, sem_b):
    c = lax.axis_index("c")
    s = lax.axis_index("s")
    wid = s * NC + c

    pltpu.sync_copy(gidx_hbm.at[wid], gidx_v)
    pltpu.sync_copy(src_hbm.at[wid], src_v)
    pltpu.sync_copy(dst_hbm.at[wid], dst_v)
    pltpu.sync_copy(cs_hbm.at[wid], cs_v)
    pltpu.sync_copy(cd_hbm.at[wid], cd_v)
    pltpu.sync_copy(nb_hbm, nb_v)
    pltpu.sync_copy(mu_hbm, mu_v)

    def remap(i, _):
        half = i // (2 * P)
        ch = i % (2 * P)
        off = half * HLEN + 2 * HGE + ch * 16
        v = gidx_v[pl.ds(off, 16)]
        gidx_v[pl.ds(off, 16)] = jnp.where(v == 0, N_NODES, v)
        return 0

    lax.fori_loop(0, NBLK * 2 * 2 * P, remap, 0)

    mu_vec = mu_v[...]
    lane = lax.broadcasted_iota(jnp.int32, (16,), 0)

    def launch_half(k, half, rows, sem):
        base = k * 2 * HLEN + half * HLEN
        return [pltpu.async_copy(
            h_hbm.at[gidx_v.at[pl.ds(base + q * QI, QI)]],
            rows.at[pl.ds(q * QI, QI)], sem) for q in range(3)]

    def compute_half(half, rows, csq, cdq, scores):
        for e in range(HGE):
            ge = half * HGE + e

            def chunk(ch, accs, e=e):
                a0, a1, a2 = accs
                sl = pl.ds(ch * 16, 16)
                hs = rows[e, sl]
                hd = rows[HGE + e, sl]
                S = rows[2 * HGE + e * P, sl]
                Dv = rows[2 * HGE + HGE * P + e * P, sl]
                for p in range(1, P):
                    S = S + rows[2 * HGE + e * P + p, sl]
                    Dv = Dv + rows[2 * HGE + HGE * P + e * P + p, sl]
                return (a0 + hs * hd, a1 + hd * S, a2 + hs * Dv)

            z = jnp.zeros((16,), _f32)
            a0, a1, a2 = lax.fori_loop(0, D // 16, chunk, (z, z, z))
            s_e = jnp.sum(a0) + csq[ge] * jnp.sum(a1) + cdq[ge] * jnp.sum(a2)
            scores = jnp.where(lane == ge, s_e, scores)
        return scores

    def block(k, _):
        cps_a = launch_half(k, 0, rows_a, sem_a)
        cps_b = launch_half(k, 1, rows_b, sem_b)

        csv = cs_v[pl.ds(k * GE, GE)]
        cdv = cd_v[pl.ds(k * GE, GE)]
        csq = csv * csv
        cdq = cdv * cdv

        for cp in cps_a:
            cp.wait()
        scores = compute_half(0, rows_a, csq, cdq, jnp.zeros((16,), _f32))
        for cp in cps_b:
            cp.wait()
        scores = compute_half(1, rows_b, csq, cdq, scores)

        srcv = src_v[pl.ds(k * GE, GE)]
        dstv = dst_v[pl.ds(k * GE, GE)]
        nbs = plsc.load_gather(nb_v, [srcv + 1])
        nbd = plsc.load_gather(nb_v, [dstv + 1])
        out_v[pl.ds(k * GE, GE)] = scores + mu_vec + nbs + nbd
        return 0

    lax.fori_loop(0, NBLK, block, 0)
    pltpu.sync_copy(out_v, score_hbm.at[pl.ds(wid * BPW, BPW)])


_score_call = functools.partial(
    pl.kernel,
    out_type=jax.ShapeDtypeStruct((B,), _f32),
    mesh=plsc.VectorSubcoreMesh(core_axis_name="c", subcore_axis_name="s",
                                num_cores=NC, num_subcores=NS),
    compiler_params=pltpu.CompilerParams(needs_layout_passes=False),
    scratch_types=[
        pltpu.VMEM((GIDX_W,), jnp.int32),
        pltpu.VMEM((BPW,), jnp.int32),
        pltpu.VMEM((BPW,), jnp.int32),
        pltpu.VMEM((BPW,), _f32),
        pltpu.VMEM((BPW,), _f32),
        pltpu.VMEM((NB_PAD,), _f32),
        pltpu.VMEM((16,), _f32),
        pltpu.VMEM((HLEN, D), _f32),
        pltpu.VMEM((HLEN, D), _f32),
        pltpu.VMEM((BPW,), _f32),
        pltpu.SemaphoreType.DMA,
        pltpu.SemaphoreType.DMA,
    ],
)(_score_body)


def _kernel_debug_p1(x, edge_index, src, dst, s2d, s2dc, d2s, d2sc, W_self,
                     W_neigh, b, node_biases, mu):
    pad_e = NW * EPWP - N_EDGES
    srcL = jnp.concatenate(
        [edge_index[0], jnp.zeros((pad_e,), jnp.int32)]).reshape(NW, NCHP, ECH)
    trash = TRASH + (jnp.arange(pad_e, dtype=jnp.int32) % (NPAD - TRASH))
    dstL = jnp.concatenate(
        [edge_index[1], trash]).reshape(NW, NCHP, ECH)
    msum = _edge_call(x, srcL, dstL)
    dcnt = _deg_call(edge_index[1].reshape(NW, EPW))
    msgs = (msum[0] + msum[1])[:N_NODES]
    deg = dcnt.sum(axis=0)[:N_NODES]
    agg = msgs / jnp.clip(deg, 1.0)[:, None]
    h_output = jax.nn.relu(x @ W_self + agg @ W_neigh + b)
    h_src = h_output[src]
    h_dst = h_output[dst]
    s2d_imp = h_output[s2d] * (s2d != 0)[..., None].astype(_f32)
    d2s_imp = h_output[d2s] * (d2s != 0)[..., None].astype(_f32)
    s2d_term = s2dc * s2dc * (h_dst * s2d_imp.sum(axis=1)).sum(axis=1)
    d2s_term = d2sc * d2sc * (h_src * d2s_imp.sum(axis=1)).sum(axis=1)
    implicit = s2d_term + d2s_term
    return (mu + (h_src * h_dst).sum(axis=1) + node_biases[src + 1]
            + node_biases[dst + 1] + implicit)


def kernel(x, edge_index, src, dst, s2d, s2dc, d2s, d2sc, W_self, W_neigh, b,
           node_biases, mu):
    pad_e = NW * EPWP - N_EDGES
    srcL = jnp.concatenate(
        [edge_index[0], jnp.zeros((pad_e,), jnp.int32)]).reshape(NW, NCHP, ECH)
    trash = TRASH + (jnp.arange(pad_e, dtype=jnp.int32) % (NPAD - TRASH))
    dstL = jnp.concatenate(
        [edge_index[1], trash]).reshape(NW, NCHP, ECH)
    msum = _edge_call(x, srcL, dstL)
    dcnt = _deg_call(edge_index[1].reshape(NW, EPW))

    x_pad = jnp.pad(x, ((0, NPAD - N_NODES), (0, 0)))
    h = _h_call(x_pad, msum[0], msum[1], dcnt, W_self, W_neigh,
                b.reshape(1, D))

    srcr = src.reshape(NW, BPW)
    dstr = dst.reshape(NW, BPW)
    src_h = src.reshape(NW, NBLK, 2, HGE)
    dst_h = dst.reshape(NW, NBLK, 2, HGE)
    s2d_h = s2d.reshape(NW, NBLK, 2, HGE * P)
    d2s_h = d2s.reshape(NW, NBLK, 2, HGE * P)
    gidx = jnp.concatenate([src_h, dst_h, s2d_h, d2s_h],
                           axis=-1).reshape(NW, GIDX_W)
    csr = s2dc.reshape(NW, BPW)
    cdr = d2sc.reshape(NW, BPW)
    nb = jnp.pad(node_biases, (0, NB_PAD - (N_NODES + 1)))
    mu_arr = jnp.broadcast_to(mu.astype(_f32), (16,))
    score = _score_call(h, gidx, srcr, dstr, csr, cdr, nb, mu_arr)
    return score


# speedup vs baseline: 1.0054x; 1.0054x over previous
import functools

import jax
import jax.numpy as jnp
from jax import lax
from jax.experimental import pallas as pl
from jax.experimental.pallas import tpu as pltpu
from jax.experimental.pallas import tpu_sc as plsc

N_NODES = 10000
D = 128
N_EDGES = 320000
B = 8192
P = 20

NC = 2
NS = 16
NW = NC * NS

EPW = N_EDGES // NW
ECH = 128
NCHP = 80
EPWP = NCHP * ECH
ICH = 16
TRASH = 10016

NPAD = 10240
STRIDE = NPAD // NS

BPW = B // NW
G = 4
NG = BPW // G

_f32 = jnp.float32


def _edge_body(x_hbm, srcL_hbm, dstL_hbm,
               msum_hbm,
               sidx_v, didx_v, rows_a, rows_b,
               msum_sh, sem_a, sem_b):
    c = lax.axis_index("c")
    s = lax.axis_index("s")
    wid = s * NC + c

    zero16 = jnp.zeros((16,), _f32)

    def zrow(i, _):
        r = i // (D // 16)
        col = (i % (D // 16)) * 16
        rows_a[r, pl.ds(col, 16)] = zero16
        return 0

    lax.fori_loop(0, ECH * (D // 16), zrow, 0)

    for k in range(STRIDE // ECH):
        off = s * STRIDE + k * ECH
        pltpu.sync_copy(rows_a, msum_sh.at[pl.ds(off, ECH)])
    plsc.subcore_barrier()

    rows = (rows_a, rows_b)
    sems = (sem_a, sem_b)
    for blk in range(NCHP // ICH):
        pltpu.sync_copy(srcL_hbm.at[wid, pl.ds(blk * ICH, ICH)], sidx_v)
        pltpu.sync_copy(dstL_hbm.at[wid, pl.ds(blk * ICH, ICH)], didx_v)
        pend = pltpu.async_copy(x_hbm.at[sidx_v.at[0]], rows[0], sems[0])
        for j in range(ICH):
            if j + 1 < ICH:
                nxt = pltpu.async_copy(x_hbm.at[sidx_v.at[j + 1]],
                                       rows[(j + 1) % 2], sems[(j + 1) % 2])
            pend.wait()
            pltpu.sync_copy(rows[j % 2], msum_sh.at[didx_v.at[j]], add=True)
            if j + 1 < ICH:
                pend = nxt
    plsc.subcore_barrier()

    for k in range(STRIDE // ECH):
        off = s * STRIDE + k * ECH
        pltpu.sync_copy(msum_sh.at[pl.ds(off, ECH)], rows[k % 2])
        pltpu.sync_copy(rows[k % 2], msum_hbm.at[c, pl.ds(off, ECH)])


_edge_call = functools.partial(
    pl.kernel,
    out_type=jax.ShapeDtypeStruct((NC, NPAD, D), _f32),
    mesh=plsc.VectorSubcoreMesh(core_axis_name="c", subcore_axis_name="s",
                                num_cores=NC, num_subcores=NS),
    compiler_params=pltpu.CompilerParams(needs_layout_passes=False),
    scratch_types=[
        pltpu.VMEM((ICH, ECH), jnp.int32),
        pltpu.VMEM((ICH, ECH), jnp.int32),
        pltpu.VMEM((ECH, D), _f32),
        pltpu.VMEM((ECH, D), _f32),
        pltpu.VMEM_SHARED((NPAD, D), _f32),
        pltpu.SemaphoreType.DMA,
        pltpu.SemaphoreType.DMA,
    ],
)(_edge_body)


def _deg_body(dstF_hbm, dcnt_hbm, didx_v, deg_v):
    c = lax.axis_index("c")
    s = lax.axis_index("s")
    wid = s * NC + c

    pltpu.sync_copy(dstF_hbm.at[wid], didx_v)

    zero16 = jnp.zeros((16,), _f32)
    one16 = jnp.ones((16,), _f32)

    def zr(i, _):
        deg_v[pl.ds(i * 16, 16)] = zero16
        return 0

    lax.fori_loop(0, NPAD // 16, zr, 0)

    def chunk(i, _):
        idx = didx_v[pl.ds(i * 16, 16)]
        plsc.addupdate_scatter(deg_v, [idx], one16)
        return 0

    lax.fori_loop(0, EPW // 16, chunk, 0)

    pltpu.sync_copy(deg_v, dcnt_hbm.at[wid])


_deg_call = functools.partial(
    pl.kernel,
    out_type=jax.ShapeDtypeStruct((NW, NPAD), _f32),
    mesh=plsc.VectorSubcoreMesh(core_axis_name="c", subcore_axis_name="s",
                                num_cores=NC, num_subcores=NS),
    compiler_params=pltpu.CompilerParams(needs_layout_passes=False),
    scratch_types=[
        pltpu.VMEM((EPW,), jnp.int32),
        pltpu.VMEM((NPAD,), _f32),
    ],
)(_deg_body)


RBLK = 1024


def _h_body(x_ref, m0_ref, m1_ref, d_ref, ws_ref, wn_ref, b_ref,
            o_ref):
    i = pl.program_id(0)
    deg = jnp.sum(d_ref[...], axis=0)[:, None]
    agg = (m0_ref[...] + m1_ref[...]) / jnp.maximum(deg, 1.0)
    h = jnp.dot(x_ref[...], ws_ref[...], preferred_element_type=_f32)
    h = h + jnp.dot(agg, wn_ref[...], preferred_element_type=_f32)
    h = jnp.maximum(h + b_ref[...], 0.0)
    rows = i * RBLK + lax.broadcasted_iota(jnp.int32, (RBLK, D), 0)
    o_ref[...] = jnp.where(rows < N_NODES, h, 0.0)


_h_call = pl.pallas_call(
    _h_body,
    grid=(NPAD // RBLK,),
    in_specs=[
        pl.BlockSpec((RBLK, D), lambda i: (i, 0)),
        pl.BlockSpec((RBLK, D), lambda i: (i, 0)),
        pl.BlockSpec((RBLK, D), lambda i: (i, 0)),
        pl.BlockSpec((NW, RBLK), lambda i: (0, i)),
        pl.BlockSpec((D, D), lambda i: (0, 0)),
        pl.BlockSpec((D, D), lambda i: (0, 0)),
        pl.BlockSpec((1, D), lambda i: (0, 0)),
    ],
    out_specs=pl.BlockSpec((RBLK, D), lambda i: (i, 0)),
    out_shape=jax.ShapeDtypeStruct((NPAD, D), _f32),
)


NB_PAD = 10008
GE = 16
HGE = 8
NBLK = BPW // GE
HLEN = 2 * HGE + 2 * HGE * P
QI = HLEN // 3
GIDX_W = NBLK * 2 * HLEN


def _score_body(h_hbm, gidx_hbm, src_hbm, dst_hbm, cs_hbm, cd_hbm,
                nb_hbm, mu_hbm, score_hbm,
                gidx_v, src_v, dst_v, cs_v, cd_v, nb_v, mu_v,
                rows_a, rows_b, out_v, sem_a, sem_b):
    c = lax.axis_index("c")
    s = lax.axis_index("s")
    wid = s * NC + c

    pltpu.sync_copy(gidx_hbm.at[wid], gidx_v)
    pltpu.sync_copy(src_hbm.at[wid], src_v)
    pltpu.sync_copy(dst_hbm.at[wid], dst_v)
    pltpu.sync_copy(cs_hbm.at[wid], cs_v)
    pltpu.sync_copy(cd_hbm.at[wid], cd_v)
    pltpu.sync_copy(nb_hbm, nb_v)
    pltpu.sync_copy(mu_hbm, mu_v)

    NCHUNK = (2 * HGE * P) // 16

    def remap(i, _):
        half = i // NCHUNK
        ch = i % NCHUNK
        off = half * HLEN + 2 * HGE + ch * 16
        v = gidx_v[pl.ds(off, 16)]
        gidx_v[pl.ds(off, 16)] = jnp.where(v == 0, N_NODES, v)
        return 0

    lax.fori_loop(0, NBLK * 2 * NCHUNK, remap, 0)

    mu_vec = mu_v[...]
    lane = lax.broadcasted_iota(jnp.int32, (16,), 0)

    def launch_half(k, half, rows, sem):
        base = k * 2 * HLEN + half * HLEN
        return [pltpu.async_copy(
            h_hbm.at[gidx_v.at[pl.ds(base + q * QI, QI)]],
            rows.at[pl.ds(q * QI, QI)], sem) for q in range(3)]

    def compute_half(half, rows, csq, cdq, scores):
        for e in range(HGE):
            ge = half * HGE + e

            def chunk(ch, accs, e=e):
                a0, a1, a2 = accs
                sl = pl.ds(ch * 16, 16)
                hs = rows[e, sl]
                hd = rows[HGE + e, sl]
                S = rows[2 * HGE + e * P, sl]
                Dv = rows[2 * HGE + HGE * P + e * P, sl]
                for p in range(1, P):
                    S = S + rows[2 * HGE + e * P + p, sl]
                    Dv = Dv + rows[2 * HGE + HGE * P + e * P + p, sl]
                return (a0 + hs * hd, a1 + hd * S, a2 + hs * Dv)

            z = jnp.zeros((16,), _f32)
            a0, a1, a2 = lax.fori_loop(0, D // 16, chunk, (z, z, z))
            s_e = jnp.sum(a0) + csq[ge] * jnp.sum(a1) + cdq[ge] * jnp.sum(a2)
            scores = jnp.where(lane == ge, s_e, scores)
        return scores

    def block(k, _):
        cps_a = launch_half(k, 0, rows_a, sem_a)
        cps_b = launch_half(k, 1, rows_b, sem_b)

        csv = cs_v[pl.ds(k * GE, GE)]
        cdv = cd_v[pl.ds(k * GE, GE)]
        csq = csv * csv
        cdq = cdv * cdv

        for cp in cps_a:
            cp.wait()
        scores = compute_half(0, rows_a, csq, cdq, jnp.zeros((16,), _f32))
        for cp in cps_b:
            cp.wait()
        scores = compute_half(1, rows_b, csq, cdq, scores)

        srcv = src_v[pl.ds(k * GE, GE)]
        dstv = dst_v[pl.ds(k * GE, GE)]
        nbs = plsc.load_gather(nb_v, [srcv + 1])
        nbd = plsc.load_gather(nb_v, [dstv + 1])
        out_v[pl.ds(k * GE, GE)] = scores + mu_vec + nbs + nbd
        return 0

    lax.fori_loop(0, NBLK, block, 0)
    pltpu.sync_copy(out_v, score_hbm.at[pl.ds(wid * BPW, BPW)])


_score_call = functools.partial(
    pl.kernel,
    out_type=jax.ShapeDtypeStruct((B,), _f32),
    mesh=plsc.VectorSubcoreMesh(core_axis_name="c", subcore_axis_name="s",
                                num_cores=NC, num_subcores=NS),
    compiler_params=pltpu.CompilerParams(needs_layout_passes=False),
    scratch_types=[
        pltpu.VMEM((GIDX_W,), jnp.int32),
        pltpu.VMEM((BPW,), jnp.int32),
        pltpu.VMEM((BPW,), jnp.int32),
        pltpu.VMEM((BPW,), _f32),
        pltpu.VMEM((BPW,), _f32),
        pltpu.VMEM((NB_PAD,), _f32),
        pltpu.VMEM((16,), _f32),
        pltpu.VMEM((HLEN, D), _f32),
        pltpu.VMEM((HLEN, D), _f32),
        pltpu.VMEM((BPW,), _f32),
        pltpu.SemaphoreType.DMA,
        pltpu.SemaphoreType.DMA,
    ],
)(_score_body)


def _kernel_debug_p1(x, edge_index, src, dst, s2d, s2dc, d2s, d2sc, W_self,
                     W_neigh, b, node_biases, mu):
    pad_e = NW * EPWP - N_EDGES
    srcL = jnp.concatenate(
        [edge_index[0], jnp.zeros((pad_e,), jnp.int32)]).reshape(NW, NCHP, ECH)
    trash = TRASH + (jnp.arange(pad_e, dtype=jnp.int32) % (NPAD - TRASH))
    dstL = jnp.concatenate(
        [edge_index[1], trash]).reshape(NW, NCHP, ECH)
    msum = _edge_call(x, srcL, dstL)
    dcnt = _deg_call(edge_index[1].reshape(NW, EPW))
    msgs = (msum[0] + msum[1])[:N_NODES]
    deg = dcnt.sum(axis=0)[:N_NODES]
    agg = msgs / jnp.clip(deg, 1.0)[:, None]
    h_output = jax.nn.relu(x @ W_self + agg @ W_neigh + b)
    h_src = h_output[src]
    h_dst = h_output[dst]
    s2d_imp = h_output[s2d] * (s2d != 0)[..., None].astype(_f32)
    d2s_imp = h_output[d2s] * (d2s != 0)[..., None].astype(_f32)
    s2d_term = s2dc * s2dc * (h_dst * s2d_imp.sum(axis=1)).sum(axis=1)
    d2s_term = d2sc * d2sc * (h_src * d2s_imp.sum(axis=1)).sum(axis=1)
    implicit = s2d_term + d2s_term
    return (mu + (h_src * h_dst).sum(axis=1) + node_biases[src + 1]
            + node_biases[dst + 1] + implicit)


def kernel(x, edge_index, src, dst, s2d, s2dc, d2s, d2sc, W_self, W_neigh, b,
           node_biases, mu):
    pad_e = NW * EPWP - N_EDGES
    srcL = jnp.concatenate(
        [edge_index[0], jnp.zeros((pad_e,), jnp.int32)]).reshape(NW, NCHP, ECH)
    trash = TRASH + (jnp.arange(pad_e, dtype=jnp.int32) % (NPAD - TRASH))
    dstL = jnp.concatenate(
        [edge_index[1], trash]).reshape(NW, NCHP, ECH)
    msum = _edge_call(x, srcL, dstL)
    dcnt = _deg_call(edge_index[1].reshape(NW, EPW))

    x_pad = jnp.pad(x, ((0, NPAD - N_NODES), (0, 0)))
    h = _h_call(x_pad, msum[0], msum[1], dcnt, W_self, W_neigh,
                b.reshape(1, D))

    srcr = src.reshape(NW, BPW)
    dstr = dst.reshape(NW, BPW)
    src_h = src.reshape(NW, NBLK, 2, HGE)
    dst_h = dst.reshape(NW, NBLK, 2, HGE)
    s2d_h = s2d.reshape(NW, NBLK, 2, HGE * P)
    d2s_h = d2s.reshape(NW, NBLK, 2, HGE * P)
    gidx = jnp.concatenate([src_h, dst_h, s2d_h, d2s_h],
                           axis=-1).reshape(NW, GIDX_W)
    csr = s2dc.reshape(NW, BPW)
    cdr = d2sc.reshape(NW, BPW)
    nb = jnp.pad(node_biases, (0, NB_PAD - (N_NODES + 1)))
    mu_arr = jnp.broadcast_to(mu.astype(_f32), (16,))
    score = _score_call(h, gidx, srcr, dstr, csr, cdr, nb, mu_arr)
    return score


# speedup vs baseline: 2.1334x; 2.1220x over previous
import functools

import jax
import jax.numpy as jnp
from jax import lax
from jax.experimental import pallas as pl
from jax.experimental.pallas import tpu as pltpu
from jax.experimental.pallas import tpu_sc as plsc

N_NODES = 10000
D = 128
N_EDGES = 320000
B = 8192
P = 20

NC = 2
NS = 16
NW = NC * NS

EPW = N_EDGES // NW
ECH = 128
NCHP = 80
EPWP = NCHP * ECH
ICH = 16
TRASH = 10016

NPAD = 10240
STRIDE = NPAD // NS

BPW = B // NW
G = 4
NG = BPW // G

_f32 = jnp.float32


def _edge_body(x_hbm, srcL_hbm, dstL_hbm,
               msum_hbm,
               sidx_v, didx_v, rows_a, rows_b,
               msum_sh, sem_a, sem_b):
    c = lax.axis_index("c")
    s = lax.axis_index("s")
    wid = s * NC + c

    zero16 = jnp.zeros((16,), _f32)

    def zrow(i, _):
        r = i // (D // 16)
        col = (i % (D // 16)) * 16
        rows_a[r, pl.ds(col, 16)] = zero16
        return 0

    lax.fori_loop(0, ECH * (D // 16), zrow, 0)

    for k in range(STRIDE // ECH):
        off = s * STRIDE + k * ECH
        pltpu.sync_copy(rows_a, msum_sh.at[pl.ds(off, ECH)])
    plsc.subcore_barrier()

    rows = (rows_a, rows_b)
    sems = (sem_a, sem_b)
    for blk in range(NCHP // ICH):
        pltpu.sync_copy(srcL_hbm.at[wid, pl.ds(blk * ICH, ICH)], sidx_v)
        pltpu.sync_copy(dstL_hbm.at[wid, pl.ds(blk * ICH, ICH)], didx_v)
        pend = pltpu.async_copy(x_hbm.at[sidx_v.at[0]], rows[0], sems[0])
        for j in range(ICH):
            if j + 1 < ICH:
                nxt = pltpu.async_copy(x_hbm.at[sidx_v.at[j + 1]],
                                       rows[(j + 1) % 2], sems[(j + 1) % 2])
            pend.wait()
            pltpu.sync_copy(rows[j % 2], msum_sh.at[didx_v.at[j]], add=True)
            if j + 1 < ICH:
                pend = nxt
    plsc.subcore_barrier()

    for k in range(STRIDE // ECH):
        off = s * STRIDE + k * ECH
        pltpu.sync_copy(msum_sh.at[pl.ds(off, ECH)], rows[k % 2])
        pltpu.sync_copy(rows[k % 2], msum_hbm.at[c, pl.ds(off, ECH)])


_edge_call = functools.partial(
    pl.kernel,
    out_type=jax.ShapeDtypeStruct((NC, NPAD, D), _f32),
    mesh=plsc.VectorSubcoreMesh(core_axis_name="c", subcore_axis_name="s",
                                num_cores=NC, num_subcores=NS),
    compiler_params=pltpu.CompilerParams(needs_layout_passes=False),
    scratch_types=[
        pltpu.VMEM((ICH, ECH), jnp.int32),
        pltpu.VMEM((ICH, ECH), jnp.int32),
        pltpu.VMEM((ECH, D), _f32),
        pltpu.VMEM((ECH, D), _f32),
        pltpu.VMEM_SHARED((NPAD, D), _f32),
        pltpu.SemaphoreType.DMA,
        pltpu.SemaphoreType.DMA,
    ],
)(_edge_body)


def _deg_body(dstF_hbm, dcnt_hbm, didx_v, deg_v):
    c = lax.axis_index("c")
    s = lax.axis_index("s")
    wid = s * NC + c

    pltpu.sync_copy(dstF_hbm.at[wid], didx_v)

    zero16 = jnp.zeros((16,), _f32)
    one16 = jnp.ones((16,), _f32)

    def zr(i, _):
        deg_v[pl.ds(i * 16, 16)] = zero16
        return 0

    lax.fori_loop(0, NPAD // 16, zr, 0)

    def chunk(i, _):
        idx = didx_v[pl.ds(i * 16, 16)]
        plsc.addupdate_scatter(deg_v, [idx], one16)
        return 0

    lax.fori_loop(0, EPW // 16, chunk, 0)

    pltpu.sync_copy(deg_v, dcnt_hbm.at[wid])


_deg_call = functools.partial(
    pl.kernel,
    out_type=jax.ShapeDtypeStruct((NW, NPAD), _f32),
    mesh=plsc.VectorSubcoreMesh(core_axis_name="c", subcore_axis_name="s",
                                num_cores=NC, num_subcores=NS),
    compiler_params=pltpu.CompilerParams(needs_layout_passes=False),
    scratch_types=[
        pltpu.VMEM((EPW,), jnp.int32),
        pltpu.VMEM((NPAD,), _f32),
    ],
)(_deg_body)


RBLK = 1024


def _h_body(x_ref, m0_ref, m1_ref, d_ref, ws_ref, wn_ref, b_ref,
            o_ref):
    i = pl.program_id(0)
    deg = jnp.sum(d_ref[...], axis=0)[:, None]
    agg = (m0_ref[...] + m1_ref[...]) / jnp.maximum(deg, 1.0)
    h = jnp.dot(x_ref[...], ws_ref[...], preferred_element_type=_f32)
    h = h + jnp.dot(agg, wn_ref[...], preferred_element_type=_f32)
    h = jnp.maximum(h + b_ref[...], 0.0)
    rows = i * RBLK + lax.broadcasted_iota(jnp.int32, (RBLK, D), 0)
    o_ref[...] = jnp.where(rows < N_NODES, h, 0.0)


_h_call = pl.pallas_call(
    _h_body,
    grid=(NPAD // RBLK,),
    in_specs=[
        pl.BlockSpec((RBLK, D), lambda i: (i, 0)),
        pl.BlockSpec((RBLK, D), lambda i: (i, 0)),
        pl.BlockSpec((RBLK, D), lambda i: (i, 0)),
        pl.BlockSpec((NW, RBLK), lambda i: (0, i)),
        pl.BlockSpec((D, D), lambda i: (0, 0)),
        pl.BlockSpec((D, D), lambda i: (0, 0)),
        pl.BlockSpec((1, D), lambda i: (0, 0)),
    ],
    out_specs=pl.BlockSpec((RBLK, D), lambda i: (i, 0)),
    out_shape=jax.ShapeDtypeStruct((NPAD, D), _f32),
)


NB_PAD = 10008
GE = 16
HGE = 8
NBLK = BPW // GE
HLEN = 2 * HGE + 2 * HGE * P
QI = HLEN // 3
GIDX_W = NBLK * 2 * HLEN


def _score_body(h_hbm, gidx_hbm, src_hbm, dst_hbm, cs_hbm, cd_hbm,
                nb_hbm, mu_hbm, score_hbm,
                gidx_v, src_v, dst_v, cs_v, cd_v, nb_v, mu_v,
                rows_a, rows_b, out_v, sem_a, sem_b):
    c = lax.axis_index("c")
    s = lax.axis_index("s")
    wid = s * NC + c

    pltpu.sync_copy(gidx_hbm.at[wid], gidx_v)
    pltpu.sync_copy(src_hbm.at[wid], src_v)
    pltpu.sync_copy(dst_hbm.at[wid], dst_v)
    pltpu.sync_copy(cs_hbm.at[wid], cs_v)
    pltpu.sync_copy(cd_hbm.at[wid], cd_v)
    pltpu.sync_copy(nb_hbm, nb_v)
    pltpu.sync_copy(mu_hbm, mu_v)

    NCHUNK = (2 * HGE * P) // 16

    def remap(i, _):
        half = i // NCHUNK
        ch = i % NCHUNK
        off = half * HLEN + 2 * HGE + ch * 16
        v = gidx_v[pl.ds(off, 16)]
        gidx_v[pl.ds(off, 16)] = jnp.where(v == 0, N_NODES, v)
        return 0

    lax.fori_loop(0, NBLK * 2 * NCHUNK, remap, 0)

    mu_vec = mu_v[...]
    lane = lax.broadcasted_iota(jnp.int32, (16,), 0)

    def launch_half(k, half, rows, sem):
        base = k * 2 * HLEN + half * HLEN
        return [pltpu.async_copy(
            h_hbm.at[gidx_v.at[pl.ds(base + q * QI, QI)]],
            rows.at[pl.ds(q * QI, QI)], sem) for q in range(3)]

    def compute_half(half, rows, csq, cdq, scores):
        for e in range(HGE):
            ge = half * HGE + e

            def chunk(ch, accs, e=e):
                a0, a1, a2 = accs
                sl = pl.ds(ch * 16, 16)
                hs = rows[e, sl]
                hd = rows[HGE + e, sl]
                S = rows[2 * HGE + e * P, sl]
                Dv = rows[2 * HGE + HGE * P + e * P, sl]
                for p in range(1, P):
                    S = S + rows[2 * HGE + e * P + p, sl]
                    Dv = Dv + rows[2 * HGE + HGE * P + e * P + p, sl]
                return (a0 + hs * hd, a1 + hd * S, a2 + hs * Dv)

            z = jnp.zeros((16,), _f32)
            a0, a1, a2 = lax.fori_loop(0, D // 16, chunk, (z, z, z))
            s_e = jnp.sum(a0) + csq[ge] * jnp.sum(a1) + cdq[ge] * jnp.sum(a2)
            scores = jnp.where(lane == ge, s_e, scores)
        return scores

    def block(k, _):
        cps_a = launch_half(k, 0, rows_a, sem_a)
        cps_b = launch_half(k, 1, rows_b, sem_b)

        csv = cs_v[pl.ds(k * GE, GE)]
        cdv = cd_v[pl.ds(k * GE, GE)]
        csq = csv * csv
        cdq = cdv * cdv

        for cp in cps_a:
            cp.wait()
        scores = compute_half(0, rows_a, csq, cdq, jnp.zeros((16,), _f32))
        for cp in cps_b:
            cp.wait()
        scores = compute_half(1, rows_b, csq, cdq, scores)

        srcv = src_v[pl.ds(k * GE, GE)]
        dstv = dst_v[pl.ds(k * GE, GE)]
        nbs = plsc.load_gather(nb_v, [srcv + 1])
        nbd = plsc.load_gather(nb_v, [dstv + 1])
        out_v[pl.ds(k * GE, GE)] = scores + mu_vec + nbs + nbd
        return 0

    lax.fori_loop(0, NBLK, block, 0)
    pltpu.sync_copy(out_v, score_hbm.at[pl.ds(wid * BPW, BPW)])


_score_call = functools.partial(
    pl.kernel,
    out_type=jax.ShapeDtypeStruct((B,), _f32),
    mesh=plsc.VectorSubcoreMesh(core_axis_name="c", subcore_axis_name="s",
                                num_cores=NC, num_subcores=NS),
    compiler_params=pltpu.CompilerParams(needs_layout_passes=False),
    scratch_types=[
        pltpu.VMEM((GIDX_W,), jnp.int32),
        pltpu.VMEM((BPW,), jnp.int32),
        pltpu.VMEM((BPW,), jnp.int32),
        pltpu.VMEM((BPW,), _f32),
        pltpu.VMEM((BPW,), _f32),
        pltpu.VMEM((NB_PAD,), _f32),
        pltpu.VMEM((16,), _f32),
        pltpu.VMEM((HLEN, D), _f32),
        pltpu.VMEM((HLEN, D), _f32),
        pltpu.VMEM((BPW,), _f32),
        pltpu.SemaphoreType.DMA,
        pltpu.SemaphoreType.DMA,
    ],
)(_score_body)


def _kernel_debug_p1(x, edge_index, src, dst, s2d, s2dc, d2s, d2sc, W_self,
                     W_neigh, b, node_biases, mu):
    pad_e = NW * EPWP - N_EDGES
    spread = jnp.arange(pad_e, dtype=jnp.int32)
    srcL = jnp.concatenate(
        [edge_index[0], spread % N_NODES]).reshape(NW, NCHP, ECH)
    dstL = jnp.concatenate(
        [edge_index[1], TRASH + spread % (NPAD - TRASH)]).reshape(
            NW, NCHP, ECH)
    msum = _edge_call(x, srcL, dstL)
    dcnt = _deg_call(edge_index[1].reshape(NW, EPW))
    msgs = (msum[0] + msum[1])[:N_NODES]
    deg = dcnt.sum(axis=0)[:N_NODES]
    agg = msgs / jnp.clip(deg, 1.0)[:, None]
    h_output = jax.nn.relu(x @ W_self + agg @ W_neigh + b)
    h_src = h_output[src]
    h_dst = h_output[dst]
    s2d_imp = h_output[s2d] * (s2d != 0)[..., None].astype(_f32)
    d2s_imp = h_output[d2s] * (d2s != 0)[..., None].astype(_f32)
    s2d_term = s2dc * s2dc * (h_dst * s2d_imp.sum(axis=1)).sum(axis=1)
    d2s_term = d2sc * d2sc * (h_src * d2s_imp.sum(axis=1)).sum(axis=1)
    implicit = s2d_term + d2s_term
    return (mu + (h_src * h_dst).sum(axis=1) + node_biases[src + 1]
            + node_biases[dst + 1] + implicit)


def kernel(x, edge_index, src, dst, s2d, s2dc, d2s, d2sc, W_self, W_neigh, b,
           node_biases, mu):
    pad_e = NW * EPWP - N_EDGES
    spread = jnp.arange(pad_e, dtype=jnp.int32)
    srcL = jnp.concatenate(
        [edge_index[0], spread % N_NODES]).reshape(NW, NCHP, ECH)
    dstL = jnp.concatenate(
        [edge_index[1], TRASH + spread % (NPAD - TRASH)]).reshape(
            NW, NCHP, ECH)
    msum = _edge_call(x, srcL, dstL)
    dcnt = _deg_call(edge_index[1].reshape(NW, EPW))

    x_pad = jnp.pad(x, ((0, NPAD - N_NODES), (0, 0)))
    h = _h_call(x_pad, msum[0], msum[1], dcnt, W_self, W_neigh,
                b.reshape(1, D))

    srcr = src.reshape(NW, BPW)
    dstr = dst.reshape(NW, BPW)
    src_h = src.reshape(NW, NBLK, 2, HGE)
    dst_h = dst.reshape(NW, NBLK, 2, HGE)
    s2d_h = s2d.reshape(NW, NBLK, 2, HGE * P)
    d2s_h = d2s.reshape(NW, NBLK, 2, HGE * P)
    gidx = jnp.concatenate([src_h, dst_h, s2d_h, d2s_h],
                           axis=-1).reshape(NW, GIDX_W)
    csr = s2dc.reshape(NW, BPW)
    cdr = d2sc.reshape(NW, BPW)
    nb = jnp.pad(node_biases, (0, NB_PAD - (N_NODES + 1)))
    mu_arr = jnp.broadcast_to(mu.astype(_f32), (16,))
    score = _score_call(h, gidx, srcr, dstr, csr, cdr, nb, mu_arr)
    return score


# speedup vs baseline: 2.2251x; 1.0430x over previous
import functools

import jax
import jax.numpy as jnp
from jax import lax
from jax.experimental import pallas as pl
from jax.experimental.pallas import tpu as pltpu
from jax.experimental.pallas import tpu_sc as plsc

N_NODES = 10000
D = 128
N_EDGES = 320000
B = 8192
P = 20

NC = 2
NS = 16
NW = NC * NS

EPW = N_EDGES // NW
ECH = 128
NCHP = 80
EPWP = NCHP * ECH
ICH = 16
TRASH = 10016

NPAD = 10240
STRIDE = NPAD // NS

BPW = B // NW
G = 4
NG = BPW // G

_f32 = jnp.float32


def _edge_body(x_hbm, srcL_hbm, dstL_hbm,
               msum_hbm, dcnt_hbm,
               sidx_v, didx_v, rows_a, rows_b, deg_v,
               msum_sh, sem_a, sem_b):
    c = lax.axis_index("c")
    s = lax.axis_index("s")
    wid = s * NC + c

    zero16 = jnp.zeros((16,), _f32)
    one16 = jnp.ones((16,), _f32)

    def zrow(i, _):
        r = i // (D // 16)
        col = (i % (D // 16)) * 16
        rows_a[r, pl.ds(col, 16)] = zero16
        return 0

    lax.fori_loop(0, ECH * (D // 16), zrow, 0)

    def zdeg(i, _):
        deg_v[pl.ds(i * 16, 16)] = zero16
        return 0

    lax.fori_loop(0, NPAD // 16, zdeg, 0)

    for k in range(STRIDE // ECH):
        off = s * STRIDE + k * ECH
        pltpu.sync_copy(rows_a, msum_sh.at[pl.ds(off, ECH)])
    plsc.subcore_barrier()

    rows = (rows_a, rows_b)
    sems = (sem_a, sem_b)
    for blk in range(NCHP // ICH):
        pltpu.sync_copy(srcL_hbm.at[wid, pl.ds(blk * ICH, ICH)], sidx_v)
        pltpu.sync_copy(dstL_hbm.at[wid, pl.ds(blk * ICH, ICH)], didx_v)
        pend = pltpu.async_copy(x_hbm.at[sidx_v.at[0]], rows[0], sems[0])
        for j in range(ICH):
            if j + 1 < ICH:
                nxt = pltpu.async_copy(x_hbm.at[sidx_v.at[j + 1]],
                                       rows[(j + 1) % 2], sems[(j + 1) % 2])
            pend.wait()
            pltpu.sync_copy(rows[j % 2], msum_sh.at[didx_v.at[j]], add=True)
            for g in range(ECH // 16):
                idxg = didx_v[j, pl.ds(g * 16, 16)]
                plsc.addupdate_scatter(deg_v, [idxg], one16)
            if j + 1 < ICH:
                pend = nxt
    plsc.subcore_barrier()

    for k in range(STRIDE // ECH):
        off = s * STRIDE + k * ECH
        pltpu.sync_copy(msum_sh.at[pl.ds(off, ECH)], rows[k % 2])
        pltpu.sync_copy(rows[k % 2], msum_hbm.at[c, pl.ds(off, ECH)])
    pltpu.sync_copy(deg_v, dcnt_hbm.at[wid])


_edge_call = functools.partial(
    pl.kernel,
    out_type=(jax.ShapeDtypeStruct((NC, NPAD, D), _f32),
              jax.ShapeDtypeStruct((NW, NPAD), _f32)),
    mesh=plsc.VectorSubcoreMesh(core_axis_name="c", subcore_axis_name="s",
                                num_cores=NC, num_subcores=NS),
    compiler_params=pltpu.CompilerParams(needs_layout_passes=False),
    scratch_types=[
        pltpu.VMEM((ICH, ECH), jnp.int32),
        pltpu.VMEM((ICH, ECH), jnp.int32),
        pltpu.VMEM((ECH, D), _f32),
        pltpu.VMEM((ECH, D), _f32),
        pltpu.VMEM((NPAD,), _f32),
        pltpu.VMEM_SHARED((NPAD, D), _f32),
        pltpu.SemaphoreType.DMA,
        pltpu.SemaphoreType.DMA,
    ],
)(_edge_body)


def _deg_body(dstF_hbm, dcnt_hbm, didx_v, deg_v):
    c = lax.axis_index("c")
    s = lax.axis_index("s")
    wid = s * NC + c

    pltpu.sync_copy(dstF_hbm.at[wid], didx_v)

    zero16 = jnp.zeros((16,), _f32)
    one16 = jnp.ones((16,), _f32)

    def zr(i, _):
        deg_v[pl.ds(i * 16, 16)] = zero16
        return 0

    lax.fori_loop(0, NPAD // 16, zr, 0)

    def chunk(i, _):
        idx = didx_v[pl.ds(i * 16, 16)]
        plsc.addupdate_scatter(deg_v, [idx], one16)
        return 0

    lax.fori_loop(0, EPW // 16, chunk, 0)

    pltpu.sync_copy(deg_v, dcnt_hbm.at[wid])


_deg_call = functools.partial(
    pl.kernel,
    out_type=jax.ShapeDtypeStruct((NW, NPAD), _f32),
    mesh=plsc.VectorSubcoreMesh(core_axis_name="c", subcore_axis_name="s",
                                num_cores=NC, num_subcores=NS),
    compiler_params=pltpu.CompilerParams(needs_layout_passes=False),
    scratch_types=[
        pltpu.VMEM((EPW,), jnp.int32),
        pltpu.VMEM((NPAD,), _f32),
    ],
)(_deg_body)


RBLK = 1024


def _h_body(x_ref, m0_ref, m1_ref, d_ref, ws_ref, wn_ref, b_ref,
            o_ref):
    i = pl.program_id(0)
    deg = jnp.sum(d_ref[...], axis=0)[:, None]
    agg = (m0_ref[...] + m1_ref[...]) / jnp.maximum(deg, 1.0)
    h = jnp.dot(x_ref[...], ws_ref[...], preferred_element_type=_f32)
    h = h + jnp.dot(agg, wn_ref[...], preferred_element_type=_f32)
    h = jnp.maximum(h + b_ref[...], 0.0)
    rows = i * RBLK + lax.broadcasted_iota(jnp.int32, (RBLK, D), 0)
    o_ref[...] = jnp.where(rows < N_NODES, h, 0.0)


_h_call = pl.pallas_call(
    _h_body,
    grid=(NPAD // RBLK,),
    in_specs=[
        pl.BlockSpec((RBLK, D), lambda i: (i, 0)),
        pl.BlockSpec((RBLK, D), lambda i: (i, 0)),
        pl.BlockSpec((RBLK, D), lambda i: (i, 0)),
        pl.BlockSpec((NW, RBLK), lambda i: (0, i)),
        pl.BlockSpec((D, D), lambda i: (0, 0)),
        pl.BlockSpec((D, D), lambda i: (0, 0)),
        pl.BlockSpec((1, D), lambda i: (0, 0)),
    ],
    out_specs=pl.BlockSpec((RBLK, D), lambda i: (i, 0)),
    out_shape=jax.ShapeDtypeStruct((NPAD, D), _f32),
)


NB_PAD = 10008
GE = 16
HGE = 8
NBLK = BPW // GE
HLEN = 2 * HGE + 2 * HGE * P
QI = HLEN // 3
GIDX_W = NBLK * 2 * HLEN


def _score_body(h_hbm, gidx_hbm, src_hbm, dst_hbm, cs_hbm, cd_hbm,
                nb_hbm, mu_hbm, score_hbm,
                gidx_v, src_v, dst_v, cs_v, cd_v, nb_v, mu_v,
                rows_a, rows_b, out_v, sem_a, sem_b):
    c = lax.axis_index("c")
    s = lax.axis_index("s")
    wid = s * NC + c

    pltpu.sync_copy(gidx_hbm.at[wid], gidx_v)
    pltpu.sync_copy(src_hbm.at[wid], src_v)
    pltpu.sync_copy(dst_hbm.at[wid], dst_v)
    pltpu.sync_copy(cs_hbm.at[wid], cs_v)
    pltpu.sync_copy(cd_hbm.at[wid], cd_v)
    pltpu.sync_copy(nb_hbm, nb_v)
    pltpu.sync_copy(mu_hbm, mu_v)

    NCHUNK = (2 * HGE * P) // 16

    def remap(i, _):
        half = i // NCHUNK
        ch = i % NCHUNK
        off = half * HLEN + 2 * HGE + ch * 16
        v = gidx_v[pl.ds(off, 16)]
        gidx_v[pl.ds(off, 16)] = jnp.where(v == 0, N_NODES, v)
        return 0

    lax.fori_loop(0, NBLK * 2 * NCHUNK, remap, 0)

    mu_vec = mu_v[...]
    lane = lax.broadcasted_iota(jnp.int32, (16,), 0)

    def launch_half(k, half, rows, sem):
        base = k * 2 * HLEN + half * HLEN
        return [pltpu.async_copy(
            h_hbm.at[gidx_v.at[pl.ds(base + q * QI, QI)]],
            rows.at[pl.ds(q * QI, QI)], sem) for q in range(3)]

    def tree_sum(ts):
        while len(ts) > 1:
            nxt = [ts[i] + ts[i + 1] for i in range(0, len(ts) - 1, 2)]
            if len(ts) % 2:
                nxt.append(ts[-1])
            ts = nxt
        return ts[0]

    def compute_half(half, rows, csq, cdq, scores):
        for e in range(HGE):
            ge = half * HGE + e
            cs2 = csq[ge]
            cd2 = cdq[ge]

            def chunk(ch, acc, e=e, cs2=cs2, cd2=cd2):
                sl = pl.ds(ch * 16, 16)
                hs = rows[e, sl]
                hd = rows[HGE + e, sl]
                S = tree_sum([rows[2 * HGE + e * P + p, sl]
                              for p in range(P)])
                Dv = tree_sum([rows[2 * HGE + HGE * P + e * P + p, sl]
                               for p in range(P)])
                return acc + hs * hd + cs2 * (hd * S) + cd2 * (hs * Dv)

            acc = lax.fori_loop(0, D // 16, chunk, jnp.zeros((16,), _f32))
            scores = jnp.where(lane == ge, jnp.sum(acc), scores)
        return scores

    def block(k, _):
        cps_a = launch_half(k, 0, rows_a, sem_a)
        cps_b = launch_half(k, 1, rows_b, sem_b)

        csv = cs_v[pl.ds(k * GE, GE)]
        cdv = cd_v[pl.ds(k * GE, GE)]
        csq = csv * csv
        cdq = cdv * cdv

        for cp in cps_a:
            cp.wait()
        scores = compute_half(0, rows_a, csq, cdq, jnp.zeros((16,), _f32))
        for cp in cps_b:
            cp.wait()
        scores = compute_half(1, rows_b, csq, cdq, scores)

        srcv = src_v[pl.ds(k * GE, GE)]
        dstv = dst_v[pl.ds(k * GE, GE)]
        nbs = plsc.load_gather(nb_v, [srcv + 1])
        nbd = plsc.load_gather(nb_v, [dstv + 1])
        out_v[pl.ds(k * GE, GE)] = scores + mu_vec + nbs + nbd
        return 0

    lax.fori_loop(0, NBLK, block, 0)
    pltpu.sync_copy(out_v, score_hbm.at[pl.ds(wid * BPW, BPW)])


_score_call = functools.partial(
    pl.kernel,
    out_type=jax.ShapeDtypeStruct((B,), _f32),
    mesh=plsc.VectorSubcoreMesh(core_axis_name="c", subcore_axis_name="s",
                                num_cores=NC, num_subcores=NS),
    compiler_params=pltpu.CompilerParams(needs_layout_passes=False),
    scratch_types=[
        pltpu.VMEM((GIDX_W,), jnp.int32),
        pltpu.VMEM((BPW,), jnp.int32),
        pltpu.VMEM((BPW,), jnp.int32),
        pltpu.VMEM((BPW,), _f32),
        pltpu.VMEM((BPW,), _f32),
        pltpu.VMEM((NB_PAD,), _f32),
        pltpu.VMEM((16,), _f32),
        pltpu.VMEM((HLEN, D), _f32),
        pltpu.VMEM((HLEN, D), _f32),
        pltpu.VMEM((BPW,), _f32),
        pltpu.SemaphoreType.DMA,
        pltpu.SemaphoreType.DMA,
    ],
)(_score_body)


def _kernel_debug_p1(x, edge_index, src, dst, s2d, s2dc, d2s, d2sc, W_self,
                     W_neigh, b, node_biases, mu):
    pad_e = NW * EPWP - N_EDGES
    spread = jnp.arange(pad_e, dtype=jnp.int32)
    srcL = jnp.concatenate(
        [edge_index[0], spread % N_NODES]).reshape(NW, NCHP, ECH)
    dstL = jnp.concatenate(
        [edge_index[1], TRASH + spread % (NPAD - TRASH)]).reshape(
            NW, NCHP, ECH)
    msum, dcnt = _edge_call(x, srcL, dstL)
    msgs = (msum[0] + msum[1])[:N_NODES]
    deg = dcnt.sum(axis=0)[:N_NODES]
    agg = msgs / jnp.clip(deg, 1.0)[:, None]
    h_output = jax.nn.relu(x @ W_self + agg @ W_neigh + b)
    h_src = h_output[src]
    h_dst = h_output[dst]
    s2d_imp = h_output[s2d] * (s2d != 0)[..., None].astype(_f32)
    d2s_imp = h_output[d2s] * (d2s != 0)[..., None].astype(_f32)
    s2d_term = s2dc * s2dc * (h_dst * s2d_imp.sum(axis=1)).sum(axis=1)
    d2s_term = d2sc * d2sc * (h_src * d2s_imp.sum(axis=1)).sum(axis=1)
    implicit = s2d_term + d2s_term
    return (mu + (h_src * h_dst).sum(axis=1) + node_biases[src + 1]
            + node_biases[dst + 1] + implicit)


def kernel(x, edge_index, src, dst, s2d, s2dc, d2s, d2sc, W_self, W_neigh, b,
           node_biases, mu):
    pad_e = NW * EPWP - N_EDGES
    spread = jnp.arange(pad_e, dtype=jnp.int32)
    srcL = jnp.concatenate(
        [edge_index[0], spread % N_NODES]).reshape(NW, NCHP, ECH)
    dstL = jnp.concatenate(
        [edge_index[1], TRASH + spread % (NPAD - TRASH)]).reshape(
            NW, NCHP, ECH)
    msum, dcnt = _edge_call(x, srcL, dstL)

    x_pad = jnp.pad(x, ((0, NPAD - N_NODES), (0, 0)))
    h = _h_call(x_pad, msum[0], msum[1], dcnt, W_self, W_neigh,
                b.reshape(1, D))

    srcr = src.reshape(NW, BPW)
    dstr = dst.reshape(NW, BPW)
    src_h = src.reshape(NW, NBLK, 2, HGE)
    dst_h = dst.reshape(NW, NBLK, 2, HGE)
    s2d_h = s2d.reshape(NW, NBLK, 2, HGE * P)
    d2s_h = d2s.reshape(NW, NBLK, 2, HGE * P)
    gidx = jnp.concatenate([src_h, dst_h, s2d_h, d2s_h],
                           axis=-1).reshape(NW, GIDX_W)
    csr = s2dc.reshape(NW, BPW)
    cdr = d2sc.reshape(NW, BPW)
    nb = jnp.pad(node_biases, (0, NB_PAD - (N_NODES + 1)))
    mu_arr = jnp.broadcast_to(mu.astype(_f32), (16,))
    score = _score_call(h, gidx, srcr, dstr, csr, cdr, nb, mu_arr)
    return score


# speedup vs baseline: 2.6530x; 1.1923x over previous
import functools

import jax
import jax.numpy as jnp
from jax import lax
from jax.experimental import pallas as pl
from jax.experimental.pallas import tpu as pltpu
from jax.experimental.pallas import tpu_sc as plsc

N_NODES = 10000
D = 128
N_EDGES = 320000
B = 8192
P = 20

NC = 2
NS = 16
NW = NC * NS

EPW = N_EDGES // NW
ECH = 128
NCHP = 80
EPWP = NCHP * ECH
ICH = 16
TRASH = 10016

NPAD = 10240
STRIDE = NPAD // NS

BPW = B // NW
G = 4
NG = BPW // G

_f32 = jnp.float32


def _edge_body(x_hbm, srcL_hbm, dstL_hbm,
               msum_hbm, dcnt_hbm,
               sidx_v, didx_v, rows_a, rows_b, deg_v,
               msum_sh, sem_a, sem_b):
    c = lax.axis_index("c")
    s = lax.axis_index("s")
    wid = s * NC + c

    zero16 = jnp.zeros((16,), _f32)
    one16 = jnp.ones((16,), _f32)

    def zrow(i, _):
        r = i // (D // 16)
        col = (i % (D // 16)) * 16
        rows_a[r, pl.ds(col, 16)] = zero16
        return 0

    lax.fori_loop(0, ECH * (D // 16), zrow, 0)

    def zdeg(i, _):
        deg_v[pl.ds(i * 16, 16)] = zero16
        return 0

    lax.fori_loop(0, NPAD // 16, zdeg, 0)

    for k in range(STRIDE // ECH):
        off = s * STRIDE + k * ECH
        pltpu.sync_copy(rows_a, msum_sh.at[pl.ds(off, ECH)])
    plsc.subcore_barrier()

    rows = (rows_a, rows_b)
    sems = (sem_a, sem_b)
    for blk in range(NCHP // ICH):
        pltpu.sync_copy(srcL_hbm.at[wid, pl.ds(blk * ICH, ICH)], sidx_v)
        pltpu.sync_copy(dstL_hbm.at[wid, pl.ds(blk * ICH, ICH)], didx_v)
        pend = pltpu.async_copy(x_hbm.at[sidx_v.at[0]], rows[0], sems[0])
        for j in range(ICH):
            if j + 1 < ICH:
                nxt = pltpu.async_copy(x_hbm.at[sidx_v.at[j + 1]],
                                       rows[(j + 1) % 2], sems[(j + 1) % 2])
            pend.wait()
            pltpu.sync_copy(rows[j % 2], msum_sh.at[didx_v.at[j]], add=True)
            for g in range(ECH // 16):
                idxg = didx_v[j, pl.ds(g * 16, 16)]
                plsc.addupdate_scatter(deg_v, [idxg], one16)
            if j + 1 < ICH:
                pend = nxt
    plsc.subcore_barrier()

    for k in range(STRIDE // ECH):
        off = s * STRIDE + k * ECH
        pltpu.sync_copy(msum_sh.at[pl.ds(off, ECH)], rows[k % 2])
        pltpu.sync_copy(rows[k % 2], msum_hbm.at[c, pl.ds(off, ECH)])
    pltpu.sync_copy(deg_v, dcnt_hbm.at[wid])


_edge_call = functools.partial(
    pl.kernel,
    out_type=(jax.ShapeDtypeStruct((NC, NPAD, D), _f32),
              jax.ShapeDtypeStruct((NW, NPAD), _f32)),
    mesh=plsc.VectorSubcoreMesh(core_axis_name="c", subcore_axis_name="s",
                                num_cores=NC, num_subcores=NS),
    compiler_params=pltpu.CompilerParams(needs_layout_passes=False),
    scratch_types=[
        pltpu.VMEM((ICH, ECH), jnp.int32),
        pltpu.VMEM((ICH, ECH), jnp.int32),
        pltpu.VMEM((ECH, D), _f32),
        pltpu.VMEM((ECH, D), _f32),
        pltpu.VMEM((NPAD,), _f32),
        pltpu.VMEM_SHARED((NPAD, D), _f32),
        pltpu.SemaphoreType.DMA,
        pltpu.SemaphoreType.DMA,
    ],
)(_edge_body)


def _deg_body(dstF_hbm, dcnt_hbm, didx_v, deg_v):
    c = lax.axis_index("c")
    s = lax.axis_index("s")
    wid = s * NC + c

    pltpu.sync_copy(dstF_hbm.at[wid], didx_v)

    zero16 = jnp.zeros((16,), _f32)
    one16 = jnp.ones((16,), _f32)

    def zr(i, _):
        deg_v[pl.ds(i * 16, 16)] = zero16
        return 0

    lax.fori_loop(0, NPAD // 16, zr, 0)

    def chunk(i, _):
        idx = didx_v[pl.ds(i * 16, 16)]
        plsc.addupdate_scatter(deg_v, [idx], one16)
        return 0

    lax.fori_loop(0, EPW // 16, chunk, 0)

    pltpu.sync_copy(deg_v, dcnt_hbm.at[wid])


_deg_call = functools.partial(
    pl.kernel,
    out_type=jax.ShapeDtypeStruct((NW, NPAD), _f32),
    mesh=plsc.VectorSubcoreMesh(core_axis_name="c", subcore_axis_name="s",
                                num_cores=NC, num_subcores=NS),
    compiler_params=pltpu.CompilerParams(needs_layout_passes=False),
    scratch_types=[
        pltpu.VMEM((EPW,), jnp.int32),
        pltpu.VMEM((NPAD,), _f32),
    ],
)(_deg_body)


RBLK = 1024


def _h_body(x_ref, m0_ref, m1_ref, d_ref, ws_ref, wn_ref, b_ref,
            o_ref):
    i = pl.program_id(0)
    deg = jnp.sum(d_ref[...], axis=0)[:, None]
    agg = (m0_ref[...] + m1_ref[...]) / jnp.maximum(deg, 1.0)
    h = jnp.dot(x_ref[...], ws_ref[...], preferred_element_type=_f32)
    h = h + jnp.dot(agg, wn_ref[...], preferred_element_type=_f32)
    h = jnp.maximum(h + b_ref[...], 0.0)
    rows = i * RBLK + lax.broadcasted_iota(jnp.int32, (RBLK, D), 0)
    o_ref[...] = jnp.where(rows < N_NODES, h, 0.0)


_h_call = pl.pallas_call(
    _h_body,
    grid=(NPAD // RBLK,),
    in_specs=[
        pl.BlockSpec((RBLK, D), lambda i: (i, 0)),
        pl.BlockSpec((RBLK, D), lambda i: (i, 0)),
        pl.BlockSpec((RBLK, D), lambda i: (i, 0)),
        pl.BlockSpec((NW, RBLK), lambda i: (0, i)),
        pl.BlockSpec((D, D), lambda i: (0, 0)),
        pl.BlockSpec((D, D), lambda i: (0, 0)),
        pl.BlockSpec((1, D), lambda i: (0, 0)),
    ],
    out_specs=pl.BlockSpec((RBLK, D), lambda i: (i, 0)),
    out_shape=jax.ShapeDtypeStruct((NPAD, D), _f32),
)


NB_PAD = 10008
GE = 16
HGE = 8
NBLK = BPW // GE
HLEN = 2 * HGE + 2 * HGE * P
QI = HLEN // 3
GIDX_W = NBLK * 2 * HLEN


def _score_body(h_hbm, gidx_hbm, src_hbm, dst_hbm, cs_hbm, cd_hbm,
                nb_hbm, mu_hbm, score_hbm,
                gidx_v, src_v, dst_v, cs_v, cd_v, nb_v, mu_v,
                rows_a, rows_b, out_v, sem_a, sem_b):
    c = lax.axis_index("c")
    s = lax.axis_index("s")
    wid = s * NC + c

    pltpu.sync_copy(gidx_hbm.at[wid], gidx_v)
    pltpu.sync_copy(src_hbm.at[wid], src_v)
    pltpu.sync_copy(dst_hbm.at[wid], dst_v)
    pltpu.sync_copy(cs_hbm.at[wid], cs_v)
    pltpu.sync_copy(cd_hbm.at[wid], cd_v)
    pltpu.sync_copy(nb_hbm, nb_v)
    pltpu.sync_copy(mu_hbm, mu_v)

    NCHUNK = (2 * HGE * P) // 16

    def remap(i, _):
        half = i // NCHUNK
        ch = i % NCHUNK
        off = half * HLEN + 2 * HGE + ch * 16
        v = gidx_v[pl.ds(off, 16)]
        gidx_v[pl.ds(off, 16)] = jnp.where(v == 0, N_NODES, v)
        return 0

    lax.fori_loop(0, NBLK * 2 * NCHUNK, remap, 0)

    mu_vec = mu_v[...]
    lane = lax.broadcasted_iota(jnp.int32, (16,), 0)

    def launch_half(k, half, rows, sem):
        base = k * 2 * HLEN + half * HLEN
        return [pltpu.async_copy(
            h_hbm.at[gidx_v.at[pl.ds(base + q * QI, QI)]],
            rows.at[pl.ds(q * QI, QI)], sem) for q in range(3)]

    def tree_sum(ts):
        while len(ts) > 1:
            nxt = [ts[i] + ts[i + 1] for i in range(0, len(ts) - 1, 2)]
            if len(ts) % 2:
                nxt.append(ts[-1])
            ts = nxt
        return ts[0]

    def compute_half(half, rows, csq, cdq, scores):
        for e in range(HGE):
            ge = half * HGE + e
            cs2 = csq[ge]
            cd2 = cdq[ge]

            def chunk(ch, acc, e=e, cs2=cs2, cd2=cd2):
                sl = pl.ds(ch * 16, 16)
                hs = rows[e, sl]
                hd = rows[HGE + e, sl]
                S = tree_sum([rows[2 * HGE + e * P + p, sl]
                              for p in range(P)])
                Dv = tree_sum([rows[2 * HGE + HGE * P + e * P + p, sl]
                               for p in range(P)])
                return acc + hs * hd + cs2 * (hd * S) + cd2 * (hs * Dv)

            acc = lax.fori_loop(0, D // 16, chunk, jnp.zeros((16,), _f32))
            scores = jnp.where(lane == ge, jnp.sum(acc), scores)
        return scores

    def drain(rows, sem):
        for q in range(3):
            pltpu.make_async_copy(h_hbm.at[pl.ds(0, QI)],
                                  rows.at[pl.ds(q * QI, QI)], sem).wait()

    def block(k, _):
        launch_half(k, 1, rows_b, sem_b)

        csv = cs_v[pl.ds(k * GE, GE)]
        cdv = cd_v[pl.ds(k * GE, GE)]
        csq = csv * csv
        cdq = cdv * cdv

        drain(rows_a, sem_a)
        scores = compute_half(0, rows_a, csq, cdq, jnp.zeros((16,), _f32))
        kn = jnp.minimum(k + 1, NBLK - 1)
        launch_half(kn, 0, rows_a, sem_a)
        drain(rows_b, sem_b)
        scores = compute_half(1, rows_b, csq, cdq, scores)

        srcv = src_v[pl.ds(k * GE, GE)]
        dstv = dst_v[pl.ds(k * GE, GE)]
        nbs = plsc.load_gather(nb_v, [srcv + 1])
        nbd = plsc.load_gather(nb_v, [dstv + 1])
        out_v[pl.ds(k * GE, GE)] = scores + mu_vec + nbs + nbd
        return 0

    launch_half(0, 0, rows_a, sem_a)
    lax.fori_loop(0, NBLK, block, 0)
    drain(rows_a, sem_a)
    pltpu.sync_copy(out_v, score_hbm.at[pl.ds(wid * BPW, BPW)])


_score_call = functools.partial(
    pl.kernel,
    out_type=jax.ShapeDtypeStruct((B,), _f32),
    mesh=plsc.VectorSubcoreMesh(core_axis_name="c", subcore_axis_name="s",
                                num_cores=NC, num_subcores=NS),
    compiler_params=pltpu.CompilerParams(needs_layout_passes=False),
    scratch_types=[
        pltpu.VMEM((GIDX_W,), jnp.int32),
        pltpu.VMEM((BPW,), jnp.int32),
        pltpu.VMEM((BPW,), jnp.int32),
        pltpu.VMEM((BPW,), _f32),
        pltpu.VMEM((BPW,), _f32),
        pltpu.VMEM((NB_PAD,), _f32),
        pltpu.VMEM((16,), _f32),
        pltpu.VMEM((HLEN, D), _f32),
        pltpu.VMEM((HLEN, D), _f32),
        pltpu.VMEM((BPW,), _f32),
        pltpu.SemaphoreType.DMA,
        pltpu.SemaphoreType.DMA,
    ],
)(_score_body)


def _kernel_debug_p1(x, edge_index, src, dst, s2d, s2dc, d2s, d2sc, W_self,
                     W_neigh, b, node_biases, mu):
    pad_e = NW * EPWP - N_EDGES
    spread = jnp.arange(pad_e, dtype=jnp.int32)
    srcL = jnp.concatenate(
        [edge_index[0], spread % N_NODES]).reshape(NW, NCHP, ECH)
    dstL = jnp.concatenate(
        [edge_index[1], TRASH + spread % (NPAD - TRASH)]).reshape(
            NW, NCHP, ECH)
    msum, dcnt = _edge_call(x, srcL, dstL)
    msgs = (msum[0] + msum[1])[:N_NODES]
    deg = dcnt.sum(axis=0)[:N_NODES]
    agg = msgs / jnp.clip(deg, 1.0)[:, None]
    h_output = jax.nn.relu(x @ W_self + agg @ W_neigh + b)
    h_src = h_output[src]
    h_dst = h_output[dst]
    s2d_imp = h_output[s2d] * (s2d != 0)[..., None].astype(_f32)
    d2s_imp = h_output[d2s] * (d2s != 0)[..., None].astype(_f32)
    s2d_term = s2dc * s2dc * (h_dst * s2d_imp.sum(axis=1)).sum(axis=1)
    d2s_term = d2sc * d2sc * (h_src * d2s_imp.sum(axis=1)).sum(axis=1)
    implicit = s2d_term + d2s_term
    return (mu + (h_src * h_dst).sum(axis=1) + node_biases[src + 1]
            + node_biases[dst + 1] + implicit)


def kernel(x, edge_index, src, dst, s2d, s2dc, d2s, d2sc, W_self, W_neigh, b,
           node_biases, mu):
    pad_e = NW * EPWP - N_EDGES
    spread = jnp.arange(pad_e, dtype=jnp.int32)
    srcL = jnp.concatenate(
        [edge_index[0], spread % N_NODES]).reshape(NW, NCHP, ECH)
    dstL = jnp.concatenate(
        [edge_index[1], TRASH + spread % (NPAD - TRASH)]).reshape(
            NW, NCHP, ECH)
    msum, dcnt = _edge_call(x, srcL, dstL)

    x_pad = jnp.pad(x, ((0, NPAD - N_NODES), (0, 0)))
    h = _h_call(x_pad, msum[0], msum[1], dcnt, W_self, W_neigh,
                b.reshape(1, D))

    srcr = src.reshape(NW, BPW)
    dstr = dst.reshape(NW, BPW)
    src_h = src.reshape(NW, NBLK, 2, HGE)
    dst_h = dst.reshape(NW, NBLK, 2, HGE)
    s2d_h = s2d.reshape(NW, NBLK, 2, HGE * P)
    d2s_h = d2s.reshape(NW, NBLK, 2, HGE * P)
    gidx = jnp.concatenate([src_h, dst_h, s2d_h, d2s_h],
                           axis=-1).reshape(NW, GIDX_W)
    csr = s2dc.reshape(NW, BPW)
    cdr = d2sc.reshape(NW, BPW)
    nb = jnp.pad(node_biases, (0, NB_PAD - (N_NODES + 1)))
    mu_arr = jnp.broadcast_to(mu.astype(_f32), (16,))
    score = _score_call(h, gidx, srcr, dstr, csr, cdr, nb, mu_arr)
    return score
